# baseline (speedup 1.0000x reference)
import jax
import jax.numpy as jnp
from jax.experimental import pallas as pl
from jax.experimental.pallas import tpu as pltpu

_B = 256; _T = 8; _D = 512; _NL = 2; _E = 8; _F = 1024
_VOCAB = 119; _FEAT = 200; _PBF = 16; _PB_SCALE = 10.0
_N = _B * _T


def _ln_rows(x, g, b):
    mu = jnp.mean(x, axis=-1, keepdims=True)
    xc = x - mu
    var = jnp.mean(xc * xc, axis=-1, keepdims=True)
    return xc * jax.lax.rsqrt(var + 1e-5) * g + b


def _enc_body(pbW_s, pbb_s, pba_s,
              src_ref, fri_ref, frj_ref, cbfv_ref, Wm_ref, bm_ref,
              kw_ref, kb_ref, vw_ref, vb_ref, ow_ref, ob_ref,
              g1_ref, b1_ref, gw_ref, gb_ref,
              ew1_ref, eb1_ref, ew2_ref, eb2_ref, g2_ref, b2_ref,
              out_ref,
              x_ref, pb_ref, w_ref, acc_ref):
    i = pl.program_id(0)
    j = pl.program_id(1)

    @pl.when(jnp.logical_and(i == 0, j == 0))
    def _prologue():
        cw = jnp.dot(cbfv_ref[...], Wm_ref[...], preferred_element_type=jnp.float32)
        oh = (src_ref[...] == jax.lax.broadcasted_iota(jnp.int32, (_N, _VOCAB), 1)
              ).astype(jnp.float32)
        x_ref[...] = jnp.dot(oh, cw, preferred_element_type=jnp.float32) + bm_ref[...]
        d2 = (frj_ref[...] - fri_ref[...]) * _PB_SCALE
        acc = jnp.zeros((_N, _T), jnp.float32)
        for f in range(_PBF):
            acc = acc + jnp.cos(d2 * pbW_s[f] + pbb_s[f]) * pba_s[f]
        pb_ref[...] = acc

    @pl.when(j == 0)
    def _attn():
        x = x_ref[...]
        xb = x.astype(jnp.bfloat16)
        K = jnp.dot(xb, kw_ref[0], preferred_element_type=jnp.float32) + kb_ref[0]
        V = jnp.dot(xb, vw_ref[0], preferred_element_type=jnp.float32) + vb_ref[0]
        K3 = K.reshape(_B, _T, _D)
        V3 = V.reshape(_B, _T, _D)
        pb3 = pb_ref[...].reshape(_B, _T, _T)
        wv3 = jnp.broadcast_to(jnp.sum(K3 * V3, axis=1, keepdims=True), (_B, _T, _D))
        for jj in range(_T):
            wv3 = wv3 + pb3[:, :, jj:jj + 1] * V3[:, jj:jj + 1, :]
        wv = wv3.reshape(_N, _D).astype(jnp.bfloat16)
        attn = jnp.dot(wv, ow_ref[0], preferred_element_type=jnp.float32) + ob_ref[0]
        xn = _ln_rows(x + attn, g1_ref[0], b1_ref[0])
        x_ref[...] = xn
        acc_ref[...] = xn
        logits = jnp.dot(xn, gw_ref[0], preferred_element_type=jnp.float32) + gb_ref[0]
        m = jnp.max(logits, axis=-1, keepdims=True)
        ex = jnp.exp(logits - m)
        p = ex / jnp.sum(ex, axis=-1, keepdims=True)
        lane = jax.lax.broadcasted_iota(jnp.int32, (_N, _E), 1)
        m1 = jnp.max(p, axis=-1, keepdims=True)
        i1 = jnp.min(jnp.where(p == m1, lane, _E), axis=-1, keepdims=True)
        oh1 = lane == i1
        pm = jnp.where(oh1, -1.0, p)
        m2 = jnp.max(pm, axis=-1, keepdims=True)
        i2 = jnp.min(jnp.where(pm == m2, lane, _E), axis=-1, keepdims=True)
        oh2 = lane == i2
        w_ref[...] = jnp.where(oh1, m1, 0.0) + jnp.where(oh2, m2, 0.0)

    @pl.when(j > 0)
    def _expert():
        xn = x_ref[...].astype(jnp.bfloat16)
        h = jnp.dot(xn, ew1_ref[0, 0], preferred_element_type=jnp.float32) + eb1_ref[0, 0]
        h = jnp.maximum(h, 0.0).astype(jnp.bfloat16)
        eo = jnp.dot(h, ew2_ref[0, 0], preferred_element_type=jnp.float32) + eb2_ref[0, 0]
        lane = jax.lax.broadcasted_iota(jnp.int32, (_N, _E), 1)
        we = jnp.sum(jnp.where(lane == (j - 1), w_ref[...], 0.0), axis=-1, keepdims=True)
        acc_ref[...] = acc_ref[...] + we * eo

    @pl.when(j == _E)
    def _ln2():
        y = _ln_rows(acc_ref[...], g2_ref[0], b2_ref[0])
        x_ref[...] = y

        @pl.when(i == _NL - 1)
        def _write():
            out_ref[...] = y


def _call_encoder(interpret, srcc, fri, frj, cbfv, Wm, bm2, pbW1, pbb, pba,
                  key_w, kb, val_w, vb, out_w, ob, g1, b1, gate_w, gb,
                  e_w1, eb1, e_w2, eb2, g2, b2):
    def fixed(*shape):
        return pl.BlockSpec(shape, lambda i, j: (0,) * len(shape))

    def per_layer(*shape):
        return pl.BlockSpec((1,) + shape, lambda i, j: (i,) + (0,) * len(shape))

    def per_expert(*shape):
        return pl.BlockSpec((1, 1) + shape,
                            lambda i, j: (i, jnp.maximum(j - 1, 0)) + (0,) * len(shape))

    smem = pl.BlockSpec(memory_space=pltpu.SMEM)

    return pl.pallas_call(
        _enc_body,
        grid=(_NL, _E + 1),
        in_specs=[smem, smem, smem,
                  fixed(_N, 1), fixed(_N, 1), fixed(_N, _T),
                  fixed(_VOCAB, _FEAT), fixed(_FEAT, _D), fixed(1, _D),
                  per_layer(_D, _D), per_layer(1, _D),
                  per_layer(_D, _D), per_layer(1, _D),
                  per_layer(_D, _D), per_layer(1, _D),
                  per_layer(1, _D), per_layer(1, _D),
                  per_layer(_D, _E), per_layer(1, _E),
                  per_expert(_D, _F), per_expert(1, _F),
                  per_expert(_F, _D), per_expert(1, _D),
                  per_layer(1, _D), per_layer(1, _D)],
        out_specs=pl.BlockSpec((_N, _D), lambda i, j: (0, 0)),
        out_shape=jax.ShapeDtypeStruct((_N, _D), jnp.float32),
        scratch_shapes=[pltpu.VMEM((_N, _D), jnp.float32),
                        pltpu.VMEM((_N, _T), jnp.float32),
                        pltpu.VMEM((_N, _E), jnp.float32),
                        pltpu.VMEM((_N, _D), jnp.float32)],
        compiler_params=pltpu.CompilerParams(
            dimension_semantics=("arbitrary", "arbitrary")),
        interpret=interpret,
    )(pbW1, pbb, pba, srcc, fri, frj, cbfv, Wm, bm2,
      key_w, kb, val_w, vb, out_w, ob, g1, b1, gate_w, gb,
      e_w1, eb1, e_w2, eb2, g2, b2)


def kernel(src, frac, cbfv, Wm, bm, pbW, pbb, pba, key_w, key_b, val_w, val_b,
           out_w, out_b, ln1_g, ln1_b, gate_w, gate_b, e_w1, e_b1, e_w2, e_b2,
           ln2_g, ln2_b, *, interpret=False):
    srcc = src.reshape(_N, 1).astype(jnp.int32)
    fri = frac.reshape(_N, 1)
    frj = jnp.repeat(frac, _T, axis=0)
    bf = jnp.bfloat16
    key_w = key_w.astype(bf); val_w = val_w.astype(bf); out_w = out_w.astype(bf)
    e_w1 = e_w1.astype(bf); e_w2 = e_w2.astype(bf)
    out = _call_encoder(
        interpret, srcc, fri, frj, cbfv, Wm, bm.reshape(1, _D),
        pbW.reshape(_PBF), pbb, pba,
        key_w, key_b.reshape(_NL, 1, _D), val_w, val_b.reshape(_NL, 1, _D),
        out_w, out_b.reshape(_NL, 1, _D),
        ln1_g.reshape(_NL, 1, _D), ln1_b.reshape(_NL, 1, _D),
        gate_w, gate_b.reshape(_NL, 1, _E),
        e_w1, e_b1.reshape(_NL, _E, 1, _F), e_w2, e_b2.reshape(_NL, _E, 1, _D),
        ln2_g.reshape(_NL, 1, _D), ln2_b.reshape(_NL, 1, _D))
    return out.reshape(_B, _T, _D)


# speedup vs baseline: 4.3626x; 4.3626x over previous
import jax
import jax.numpy as jnp
from jax.experimental import pallas as pl
from jax.experimental.pallas import tpu as pltpu

_B = 256; _T = 8; _D = 512; _NL = 2; _E = 8; _F = 1024
_VOCAB = 119; _FEAT = 200; _PBF = 16; _PB_SCALE = 10.0
_N = _B * _T


def _ln_rows(x, g, b):
    mu = jnp.mean(x, axis=-1, keepdims=True)
    xc = x - mu
    var = jnp.mean(xc * xc, axis=-1, keepdims=True)
    return xc * jax.lax.rsqrt(var + 1e-5) * g + b


def _enc_body(pbW_s, pbb_s, pba_s,
              src_ref, friT_ref, frjT_ref, cbfv_ref, Wm_ref, bm_ref,
              kw_ref, kb_ref, vw_ref, vb_ref, ow_ref, ob_ref,
              g1_ref, b1_ref, gw_ref, gb_ref,
              ew1_ref, eb1_ref, ew2_ref, eb2_ref, g2_ref, b2_ref,
              out_ref,
              x_ref, pb_ref, w_ref, acc_ref):
    i = pl.program_id(0)
    j = pl.program_id(1)

    @pl.when(jnp.logical_and(i == 0, j == 0))
    def _prologue():
        cw = jnp.dot(cbfv_ref[...], Wm_ref[...], preferred_element_type=jnp.float32)
        oh = (src_ref[...] == jax.lax.broadcasted_iota(jnp.int32, (_N, _VOCAB), 1)
              ).astype(jnp.float32)
        x_ref[...] = jnp.dot(oh, cw, preferred_element_type=jnp.float32) + bm_ref[...]
        dT = (frjT_ref[...] - friT_ref[...]) * _PB_SCALE
        acc = jnp.zeros((_T, _N), jnp.float32)
        for f in range(_PBF):
            acc = acc + jnp.cos(dT * pbW_s[f] + pbb_s[f]) * pba_s[f]
        pb_ref[...] = acc.T

    @pl.when(j == 0)
    def _attn():
        x = x_ref[...]
        xb = x.astype(jnp.bfloat16)
        K = jnp.dot(xb, kw_ref[0], preferred_element_type=jnp.float32) + kb_ref[0]
        V = jnp.dot(xb, vw_ref[0], preferred_element_type=jnp.float32) + vb_ref[0]
        K3 = K.reshape(_B, _T, _D)
        V3 = V.reshape(_B, _T, _D)
        pb3 = pb_ref[...].reshape(_B, _T, _T)
        wv3 = jnp.broadcast_to(jnp.sum(K3 * V3, axis=1, keepdims=True), (_B, _T, _D))
        for jj in range(_T):
            wv3 = wv3 + pb3[:, :, jj:jj + 1] * V3[:, jj:jj + 1, :]
        wv = wv3.reshape(_N, _D).astype(jnp.bfloat16)
        attn = jnp.dot(wv, ow_ref[0], preferred_element_type=jnp.float32) + ob_ref[0]
        xn = _ln_rows(x + attn, g1_ref[0], b1_ref[0])
        x_ref[...] = xn
        acc_ref[...] = xn
        logits = jnp.dot(xn, gw_ref[0], preferred_element_type=jnp.float32) + gb_ref[0]
        m = jnp.max(logits, axis=-1, keepdims=True)
        ex = jnp.exp(logits - m)
        p = ex / jnp.sum(ex, axis=-1, keepdims=True)
        lane = jax.lax.broadcasted_iota(jnp.int32, (_N, _E), 1)
        m1 = jnp.max(p, axis=-1, keepdims=True)
        i1 = jnp.min(jnp.where(p == m1, lane, _E), axis=-1, keepdims=True)
        oh1 = lane == i1
        pm = jnp.where(oh1, -1.0, p)
        m2 = jnp.max(pm, axis=-1, keepdims=True)
        i2 = jnp.min(jnp.where(pm == m2, lane, _E), axis=-1, keepdims=True)
        oh2 = lane == i2
        w_ref[...] = jnp.where(oh1, m1, 0.0) + jnp.where(oh2, m2, 0.0)

    xn = x_ref[...].astype(jnp.bfloat16)
    h = jnp.dot(xn, ew1_ref[0, 0], preferred_element_type=jnp.float32) + eb1_ref[0, 0]
    h = jnp.maximum(h, 0.0).astype(jnp.bfloat16)
    eo = jnp.dot(h, ew2_ref[0, 0], preferred_element_type=jnp.float32) + eb2_ref[0, 0]
    lane = jax.lax.broadcasted_iota(jnp.int32, (_N, _E), 1)
    we = jnp.sum(jnp.where(lane == j, w_ref[...], 0.0), axis=-1, keepdims=True)
    acc_ref[...] = acc_ref[...] + we * eo

    @pl.when(j == _E - 1)
    def _ln2():
        y = _ln_rows(acc_ref[...], g2_ref[0], b2_ref[0])
        x_ref[...] = y

        @pl.when(i == _NL - 1)
        def _write():
            out_ref[...] = y


def _call_encoder(interpret, srcc, friT, frjT, cbfv, Wm, bm2, pbW1, pbb, pba,
                  key_w, kb, val_w, vb, out_w, ob, g1, b1, gate_w, gb,
                  e_w1, eb1, e_w2, eb2, g2, b2):
    def fixed(*shape):
        return pl.BlockSpec(shape, lambda i, j: (0,) * len(shape))

    def per_layer(*shape):
        return pl.BlockSpec((1,) + shape, lambda i, j: (i,) + (0,) * len(shape))

    def per_expert(*shape):
        return pl.BlockSpec((1, 1) + shape,
                            lambda i, j: (i, j) + (0,) * len(shape))

    smem = pl.BlockSpec(memory_space=pltpu.SMEM)

    return pl.pallas_call(
        _enc_body,
        grid=(_NL, _E),
        in_specs=[smem, smem, smem,
                  fixed(_N, 1), fixed(1, _N), fixed(_T, _N),
                  fixed(_VOCAB, _FEAT), fixed(_FEAT, _D), fixed(1, _D),
                  per_layer(_D, _D), per_layer(1, _D),
                  per_layer(_D, _D), per_layer(1, _D),
                  per_layer(_D, _D), per_layer(1, _D),
                  per_layer(1, _D), per_layer(1, _D),
                  per_layer(_D, _E), per_layer(1, _E),
                  per_expert(_D, _F), per_expert(1, _F),
                  per_expert(_F, _D), per_expert(1, _D),
                  per_layer(1, _D), per_layer(1, _D)],
        out_specs=pl.BlockSpec((_N, _D), lambda i, j: (0, 0)),
        out_shape=jax.ShapeDtypeStruct((_N, _D), jnp.float32),
        scratch_shapes=[pltpu.VMEM((_N, _D), jnp.float32),
                        pltpu.VMEM((_N, _T), jnp.float32),
                        pltpu.VMEM((_N, _E), jnp.float32),
                        pltpu.VMEM((_N, _D), jnp.float32)],
        compiler_params=pltpu.CompilerParams(
            dimension_semantics=("arbitrary", "arbitrary")),
        interpret=interpret,
    )(pbW1, pbb, pba, srcc, friT, frjT, cbfv, Wm, bm2,
      key_w, kb, val_w, vb, out_w, ob, g1, b1, gate_w, gb,
      e_w1, eb1, e_w2, eb2, g2, b2)


def kernel(src, frac, cbfv, Wm, bm, pbW, pbb, pba, key_w, key_b, val_w, val_b,
           out_w, out_b, ln1_g, ln1_b, gate_w, gate_b, e_w1, e_b1, e_w2, e_b2,
           ln2_g, ln2_b, *, interpret=False):
    srcc = src.reshape(_N, 1).astype(jnp.int32)
    friT = frac.reshape(1, _N)
    frjT = jnp.repeat(frac.T, _T, axis=1)
    bf = jnp.bfloat16
    key_w = key_w.astype(bf); val_w = val_w.astype(bf); out_w = out_w.astype(bf)
    e_w1 = e_w1.astype(bf); e_w2 = e_w2.astype(bf)
    out = _call_encoder(
        interpret, srcc, friT, frjT, cbfv, Wm, bm.reshape(1, _D),
        pbW.reshape(_PBF), pbb, pba,
        key_w, key_b.reshape(_NL, 1, _D), val_w, val_b.reshape(_NL, 1, _D),
        out_w, out_b.reshape(_NL, 1, _D),
        ln1_g.reshape(_NL, 1, _D), ln1_b.reshape(_NL, 1, _D),
        gate_w, gate_b.reshape(_NL, 1, _E),
        e_w1, e_b1.reshape(_NL, _E, 1, _F), e_w2, e_b2.reshape(_NL, _E, 1, _D),
        ln2_g.reshape(_NL, 1, _D), ln2_b.reshape(_NL, 1, _D))
    return out.reshape(_B, _T, _D)
